# Initial kernel scaffold; baseline (speedup 1.0000x reference)
import jax
import jax.numpy as jnp
from jax.experimental import pallas as pl


def kernel(x, edge_index, W1, W2):
    raise NotImplementedError("write your pallas kernel here")


# speedup vs baseline: 5.0168x; 5.0168x over previous
import functools

import jax
import jax.numpy as jnp
from jax import lax
from jax.experimental import pallas as pl
from jax.experimental.pallas import tpu as pltpu
from jax.experimental.pallas import tpu_sc as plsc

N = 10000
E = 320000
D = 128
NC = 2
NS = 16
NW = NC * NS
EPW = E // NW
CH = 80
NCHUNK = EPW // CH
NP = 10112
RPT = NP // NS


def _seg_sum_body(feat_hbm, src_hbm, dst_hbm, zeros_hbm, out_hbm,
                  agg_sh, idx_s, idx_d, rows, sem):
    c = lax.axis_index("c")
    s = lax.axis_index("s")

    pltpu.sync_copy(zeros_hbm.at[pl.ds(s * RPT, RPT)],
                    agg_sh.at[pl.ds(s * RPT, RPT)])
    plsc.subcore_barrier()

    base = c * (E // NC) + s * EPW

    @pl.loop(0, NCHUNK)
    def chunk(i):
        eb = base + i * CH
        pltpu.sync_copy(src_hbm.at[pl.ds(eb, CH)], idx_s)
        pltpu.sync_copy(dst_hbm.at[pl.ds(eb, CH)], idx_d)
        pltpu.async_copy(feat_hbm.at[idx_s], rows, sem).wait()
        pltpu.sync_copy(rows, agg_sh.at[idx_d], add=True)

    plsc.subcore_barrier()

    pltpu.sync_copy(agg_sh.at[pl.ds(s * RPT, RPT)],
                    out_hbm.at[pl.ds(c * NP + s * RPT, RPT)])


_seg_sum = pl.kernel(
    _seg_sum_body,
    out_type=jax.ShapeDtypeStruct((NC * NP, D), jnp.float32),
    mesh=plsc.VectorSubcoreMesh(core_axis_name="c", subcore_axis_name="s",
                                num_cores=NC, num_subcores=NS),
    scratch_types=[
        pltpu.VMEM_SHARED((NP, D), jnp.float32),
        pltpu.VMEM((CH,), jnp.int32),
        pltpu.VMEM((CH,), jnp.int32),
        pltpu.VMEM((CH, D), jnp.float32),
        pltpu.SemaphoreType.DMA,
    ],
)

BM = 2000


def _mlp_body(relu, p0_ref, p1_ref, x_ref, w_ref, o_ref):
    acc = p0_ref[...] + p1_ref[...] + x_ref[...]
    y = lax.dot_general(acc, w_ref[...], (((1,), (1,)), ((), ())),
                        preferred_element_type=jnp.float32)
    o_ref[...] = jnp.maximum(y, 0.0) if relu else y


def _mlp(p0, p1, x, w, relu):
    dout = w.shape[0]
    return pl.pallas_call(
        functools.partial(_mlp_body, relu),
        grid=(N // BM,),
        in_specs=[
            pl.BlockSpec((BM, D), lambda i: (i, 0)),
            pl.BlockSpec((BM, D), lambda i: (i, 0)),
            pl.BlockSpec((BM, D), lambda i: (i, 0)),
            pl.BlockSpec((dout, D), lambda i: (0, 0)),
        ],
        out_specs=pl.BlockSpec((BM, dout), lambda i: (i, 0)),
        out_shape=jax.ShapeDtypeStruct((N, dout), jnp.float32),
    )(p0, p1, x, w)


@jax.jit
def kernel(x, edge_index, W1, W2):
    src = edge_index[0]
    dst = edge_index[1]
    zeros = jnp.zeros((NP, D), jnp.float32)
    p1 = _seg_sum(x, src, dst, zeros)
    h = _mlp(p1[:N], p1[NP:NP + N], x, W1, relu=True)
    p2 = _seg_sum(h, src, dst, zeros)
    out = _mlp(p2[:N], p2[NP:NP + N], h, W2, relu=False)
    return out
